# Initial kernel scaffold; baseline (speedup 1.0000x reference)
import jax
import jax.numpy as jnp
from jax.experimental import pallas as pl


def kernel(x, params, idx):
    raise NotImplementedError("write your pallas kernel here")


# speedup vs baseline: 1.2721x; 1.2721x over previous
import functools

import jax
import jax.numpy as jnp
from jax import lax
from jax.experimental import pallas as pl
from jax.experimental.pallas import tpu as pltpu
from jax.experimental.pallas import tpu_sc as plsc

_NS = [40962, 10242, 2562, 642, 162, 42]
_CHS = [3, 32, 64, 128, 256, 512]
_CHP = [16, 32, 64, 128, 256, 512]
_UPS = [(1, 42, 162, 4), (2, 162, 642, 3), (3, 642, 2562, 2),
        (4, 2562, 10242, 1), (5, 10242, 40962, 0)]
_UOC = {1: 256, 2: 128, 3: 64, 4: 32, 5: 3}

_NCORES = 2
_NSUB = 16
_NW = _NCORES * _NSUB

@functools.cache
def _sc_mesh():
    return plsc.VectorSubcoreMesh(core_axis_name="c", subcore_axis_name="s",
                                  num_cores=_NCORES, num_subcores=_NSUB)


def _ceil_to(x, m):
    return -(-x // m) * m


@functools.cache
def _gather_call(C, M_pad):
    assert M_pad % (8 * _NW) == 0 and C % 8 == 0
    bpw = M_pad // _NW
    cap = max(8, min(1024, (32768 // C) // 8 * 8))
    n_chunks = -(-bpw // cap)
    chunk = _ceil_to(-(-bpw // n_chunks), 8)
    sizes = []
    off = 0
    while off < bpw:
        sz = min(chunk, bpw - off)
        sizes.append((off, sz))
        off += sz

    @functools.partial(
        pl.kernel,
        out_type=jax.ShapeDtypeStruct((M_pad, C), jnp.float32),
        mesh=_sc_mesh(),
        scratch_types=[
            pltpu.VMEM((chunk,), jnp.int32),
            pltpu.VMEM((chunk, C), jnp.float32),
            pltpu.SemaphoreType.DMA,
        ],
        compiler_params=pltpu.CompilerParams(use_tc_tiling_on_sc=False),
    )
    def gk(table_hbm, idx_hbm, out_hbm, idx_v, rows_v, sem):
        wid = lax.axis_index("s") * _NCORES + lax.axis_index("c")
        base = wid * bpw
        for off_, sz in sizes:
            pltpu.sync_copy(idx_hbm.at[pl.ds(base + off_, sz)],
                            idx_v.at[pl.ds(0, sz)])
            handles = []
            s0 = 0
            while s0 < sz:
                s = min(128, sz - s0)
                handles.append(pltpu.async_copy(
                    table_hbm.at[idx_v.at[pl.ds(s0, s)]],
                    rows_v.at[pl.ds(s0, s)], sem))
                s0 += s
            for h in handles:
                h.wait()
            pltpu.sync_copy(rows_v.at[pl.ds(0, sz)],
                            out_hbm.at[pl.ds(base + off_, sz)])

    return gk


def _gather(table, idx):
    M = idx.shape[0]
    C = table.shape[1]
    M_pad = _ceil_to(M, 8 * _NW)
    idxp = jnp.pad(idx, (0, M_pad - M)) if M_pad != M else idx
    out = _gather_call(C, M_pad)(table, idxp)
    return out[:M] if M_pad != M else out


def _dot(a, b, dims):
    return lax.dot_general(a, b, (dims, ((), ())),
                           precision=lax.Precision.HIGHEST,
                           preferred_element_type=jnp.float32)


def _conv_bn_body(sigmoid, m_ref, w_ref, b_ref, g_ref, be_ref, o_ref):
    h2 = _dot(m_ref[:], w_ref[:], ((1,), (1,))) + b_ref[:]
    mu = jnp.mean(h2, axis=0, keepdims=True)
    var = jnp.mean((h2 - mu) ** 2, axis=0, keepdims=True)
    h2 = (h2 - mu) * lax.rsqrt(var + 1e-5) * g_ref[:] + be_ref[:]
    h2 = jnp.where(h2 >= 0, h2, 0.2 * h2)
    if sigmoid:
        h2 = 1.0 / (1.0 + jnp.exp(-h2))
    o_ref[:] = h2


_CONV_BLOCK = 1024


def _mm_stats_body(n, bn, m_ref, w_ref, b_ref, h2_ref, st_ref):
    i = pl.program_id(0)
    h2 = _dot(m_ref[:], w_ref[:], ((1,), (1,))) + b_ref[:]
    h2_ref[:] = h2
    rows = lax.broadcasted_iota(jnp.int32, (bn, 1), 0)
    valid = rows < (n - i * bn)
    h2m = jnp.where(valid, h2, 0.0)
    s = jnp.sum(h2m, axis=0, keepdims=True)
    s2 = jnp.sum(h2m * h2m, axis=0, keepdims=True)

    @pl.when(i == 0)
    def _():
        st_ref[:] = jnp.zeros_like(st_ref)

    st_ref[0:1, :] += s
    st_ref[1:2, :] += s2


def _bn_act_body(n, sigmoid, h2_ref, st_ref, g_ref, be_ref, o_ref):
    mu = st_ref[0:1, :] / n
    var = st_ref[1:2, :] / n - mu * mu
    h2 = (h2_ref[:] - mu) * lax.rsqrt(var + 1e-5) * g_ref[:] + be_ref[:]
    h2 = jnp.where(h2 >= 0, h2, 0.2 * h2)
    if sigmoid:
        h2 = 1.0 / (1.0 + jnp.exp(-h2))
    o_ref[:] = h2


def _conv_bn(mat, W, b, g, be, sigmoid=False):
    n, kdim = mat.shape
    ocp = W.shape[0]
    b2, g2, be2 = b.reshape(1, -1), g.reshape(1, -1), be.reshape(1, -1)
    if n <= 2562:
        return pl.pallas_call(
            functools.partial(_conv_bn_body, sigmoid),
            out_shape=jax.ShapeDtypeStruct((n, ocp), jnp.float32),
        )(mat, W, b2, g2, be2)
    bn = _CONV_BLOCK
    nb = -(-n // bn)
    h2, st = pl.pallas_call(
        functools.partial(_mm_stats_body, n, bn),
        grid=(nb,),
        in_specs=[
            pl.BlockSpec((bn, kdim), lambda i: (i, 0)),
            pl.BlockSpec((ocp, kdim), lambda i: (0, 0)),
            pl.BlockSpec((1, ocp), lambda i: (0, 0)),
        ],
        out_specs=[
            pl.BlockSpec((bn, ocp), lambda i: (i, 0)),
            pl.BlockSpec((8, ocp), lambda i: (0, 0)),
        ],
        out_shape=[
            jax.ShapeDtypeStruct((n, ocp), jnp.float32),
            jax.ShapeDtypeStruct((8, ocp), jnp.float32),
        ],
    )(mat, W, b2)
    return pl.pallas_call(
        functools.partial(_bn_act_body, float(n), sigmoid),
        grid=(nb,),
        in_specs=[
            pl.BlockSpec((bn, ocp), lambda i: (i, 0)),
            pl.BlockSpec((8, ocp), lambda i: (0, 0)),
            pl.BlockSpec((1, ocp), lambda i: (0, 0)),
            pl.BlockSpec((1, ocp), lambda i: (0, 0)),
        ],
        out_specs=pl.BlockSpec((bn, ocp), lambda i: (i, 0)),
        out_shape=jax.ShapeDtypeStruct((n, ocp), jnp.float32),
    )(h2, st, g2, be2)


def _matmul_body(a_ref, b_ref, o_ref):
    o_ref[:] = _dot(a_ref[:], b_ref[:], ((1,), (0,)))


def _mean_pool(mat, S):
    n, kdim = mat.shape
    c = S.shape[1]
    if n <= 10242:
        return pl.pallas_call(
            _matmul_body,
            out_shape=jax.ShapeDtypeStruct((n, c), jnp.float32),
        )(mat, S)
    bn = _CONV_BLOCK
    nb = -(-n // bn)
    return pl.pallas_call(
        _matmul_body,
        grid=(nb,),
        in_specs=[
            pl.BlockSpec((bn, kdim), lambda i: (i, 0)),
            pl.BlockSpec((kdim, c), lambda i: (0, 0)),
        ],
        out_specs=pl.BlockSpec((bn, c), lambda i: (i, 0)),
        out_shape=jax.ShapeDtypeStruct((n, c), jnp.float32),
    )(mat, S)


def _upconv_body(h_ref, w_ref, b_ref, o_ref):
    o_ref[:] = _dot(h_ref[:], w_ref[:], ((1,), (1,))) + b_ref[:]


def _upconv(h, W, b):
    return pl.pallas_call(
        _upconv_body,
        out_shape=jax.ShapeDtypeStruct((h.shape[0], W.shape[0]), jnp.float32),
    )(h, W, b.reshape(1, -1))


def _pad_cols(a, cp):
    return a if a.shape[1] == cp else jnp.pad(a, ((0, 0), (0, cp - a.shape[1])))


def _pad_conv_params(params, name, ic, icp, oc, ocp):
    W = params[name + '_W']
    W = W.reshape(oc, 7, ic)
    W = jnp.pad(W, ((0, ocp - oc), (0, 0), (0, icp - ic))).reshape(ocp, 7 * icp)
    b = jnp.pad(params[name + '_b'], (0, ocp - oc))
    g = jnp.pad(params[name + '_g'], (0, ocp - oc))
    be = jnp.pad(params[name + '_be'], (0, ocp - oc))
    return W, b, g, be


def _mean_sel(m, c):
    return jnp.tile(jnp.eye(c, dtype=jnp.float32), (m, 1)) / float(m)


def _conv_layer(h, no, params, name, ic, icp, oc, ocp, sigmoid=False):
    n = no.shape[0] // 7
    W, b, g, be = _pad_conv_params(params, name, ic, icp, oc, ocp)
    gth = _gather(h, no)
    mat = gth.reshape(n, 7 * icp)
    return _conv_bn(mat, W, b, g, be, sigmoid=sigmoid)


def kernel(x, params, idx):
    h = _pad_cols(x, _CHP[0])
    for i in range(1, 6):
        n = _NS[i]
        icp = _CHP[i - 1]
        g = _gather(h, idx['no%d' % (i - 1)][: n * 7])
        hp = _mean_pool(g.reshape(n, 7 * icp), _mean_sel(7, icp))
        h = _conv_layer(hp, idx['no%d' % i], params, 'd%dc1' % i,
                        _CHS[i - 1], icp, _CHS[i], _CHP[i])
        h = _conv_layer(h, idx['no%d' % i], params, 'd%dc2' % i,
                        _CHS[i], _CHP[i], _CHS[i], _CHP[i])
    for (i, nf, nt, lvl) in _UPS:
        ic = _CHS[lvl + 1]
        oc = _UOC[i]
        ocp = 16 if oc == 3 else oc
        Wu = params['u%d_W' % i].reshape(7, oc, ic)
        Wu = jnp.pad(Wu, ((0, 0), (0, ocp - oc), (0, 0))).reshape(7 * ocp, ic)
        bu = jnp.pad(params['u%d_b' % i].reshape(7, oc),
                     ((0, 0), (0, ocp - oc))).reshape(7 * ocp)
        y = _upconv(h, Wu, bu)
        ytab = y.reshape(nf * 7, ocp)
        comb = jnp.concatenate(
            [jnp.repeat(idx['top%d' % nt], 2), idx['down%d' % nt]])
        g = _gather(ytab, comb)
        h = _mean_pool(g.reshape(nt, 2 * ocp), _mean_sel(2, ocp))
        h = _conv_layer(h, idx['no%d' % lvl], params, 'u%dc1' % i,
                        oc, ocp, oc, ocp)
        h = _conv_layer(h, idx['no%d' % lvl], params, 'u%dc2' % i,
                        oc, ocp, oc, ocp, sigmoid=(i == 5))
    return h[:, :3]


# speedup vs baseline: 1.4004x; 1.1008x over previous
import functools

import jax
import jax.numpy as jnp
from jax import lax
from jax.experimental import pallas as pl
from jax.experimental.pallas import tpu as pltpu
from jax.experimental.pallas import tpu_sc as plsc

_NS = [40962, 10242, 2562, 642, 162, 42]
_CHS = [3, 32, 64, 128, 256, 512]
_CHP = [16, 32, 64, 128, 256, 512]
_UPS = [(1, 42, 162, 4), (2, 162, 642, 3), (3, 642, 2562, 2),
        (4, 2562, 10242, 1), (5, 10242, 40962, 0)]
_UOC = {1: 256, 2: 128, 3: 64, 4: 32, 5: 3}

_NCORES = 2
_NSUB = 16
_NW = _NCORES * _NSUB

@functools.cache
def _sc_mesh():
    return plsc.VectorSubcoreMesh(core_axis_name="c", subcore_axis_name="s",
                                  num_cores=_NCORES, num_subcores=_NSUB)


def _ceil_to(x, m):
    return -(-x // m) * m


@functools.cache
def _gather_call(C, n_out_pad, group):
    assert n_out_pad % (8 * _NW) == 0 and C % 16 == 0
    bpw = n_out_pad // _NW
    cap = max(8, min(1024, (32768 // (C * group)) // 8 * 8))
    n_chunks = -(-bpw // cap)
    chunk = _ceil_to(-(-bpw // n_chunks), 8)
    sizes = []
    off = 0
    while off < bpw:
        sz = min(chunk, bpw - off)
        sizes.append((off, sz))
        off += sz
    scratch = [
        pltpu.VMEM((group * chunk,), jnp.int32),
        pltpu.VMEM((group * chunk, C), jnp.float32),
        pltpu.SemaphoreType.DMA,
    ]
    if group > 1:
        scratch.append(pltpu.VMEM((chunk, C), jnp.float32))

    @functools.partial(
        pl.kernel,
        out_type=jax.ShapeDtypeStruct((n_out_pad, C), jnp.float32),
        mesh=_sc_mesh(),
        scratch_types=scratch,
        compiler_params=pltpu.CompilerParams(use_tc_tiling_on_sc=False),
    )
    def gk(table_hbm, idx_hbm, out_hbm, idx_v, rows_v, sem, *maybe_out_v):
        wid = lax.axis_index("s") * _NCORES + lax.axis_index("c")
        base = wid * bpw
        for off_, sz in sizes:
            gsz = group * sz
            pltpu.sync_copy(idx_hbm.at[pl.ds((base + off_) * group, gsz)],
                            idx_v.at[pl.ds(0, gsz)])
            handles = []
            s0 = 0
            while s0 < gsz:
                s = min(128, gsz - s0)
                handles.append(pltpu.async_copy(
                    table_hbm.at[idx_v.at[pl.ds(s0, s)]],
                    rows_v.at[pl.ds(s0, s)], sem))
                s0 += s
            for h in handles:
                h.wait()
            if group == 1:
                pltpu.sync_copy(rows_v.at[pl.ds(0, sz)],
                                out_hbm.at[pl.ds(base + off_, sz)])
            else:
                out_v = maybe_out_v[0]
                inv = jnp.float32(1.0 / group)

                def body_j(j, _):
                    for c0 in range(C // 16):
                        acc = rows_v[j * group, pl.ds(c0 * 16, 16)]
                        for k in range(1, group):
                            acc = acc + rows_v[j * group + k,
                                               pl.ds(c0 * 16, 16)]
                        out_v[j, pl.ds(c0 * 16, 16)] = acc * inv
                    return 0

                lax.fori_loop(0, sz, body_j, 0)
                pltpu.sync_copy(out_v.at[pl.ds(0, sz)],
                                out_hbm.at[pl.ds(base + off_, sz)])

    return gk


def _gather(table, idx):
    M = idx.shape[0]
    C = table.shape[1]
    M_pad = _ceil_to(M, 8 * _NW)
    idxp = jnp.pad(idx, (0, M_pad - M)) if M_pad != M else idx
    out = _gather_call(C, M_pad, 1)(table, idxp)
    return out[:M] if M_pad != M else out


def _gather_mean(table, idx, group):
    n_out = idx.shape[0] // group
    C = table.shape[1]
    n_out_pad = _ceil_to(n_out, 8 * _NW)
    idxp = jnp.pad(idx, (0, group * (n_out_pad - n_out)))
    return _gather_call(C, n_out_pad, group)(table, idxp)


def _dot(a, b, dims):
    return lax.dot_general(a, b, (dims, ((), ())),
                           precision=lax.Precision.HIGHEST,
                           preferred_element_type=jnp.float32)


def _conv_bn_body(sigmoid, m_ref, w_ref, b_ref, g_ref, be_ref, o_ref):
    h2 = _dot(m_ref[:], w_ref[:], ((1,), (1,))) + b_ref[:]
    mu = jnp.mean(h2, axis=0, keepdims=True)
    var = jnp.mean((h2 - mu) ** 2, axis=0, keepdims=True)
    h2 = (h2 - mu) * lax.rsqrt(var + 1e-5) * g_ref[:] + be_ref[:]
    h2 = jnp.where(h2 >= 0, h2, 0.2 * h2)
    if sigmoid:
        h2 = 1.0 / (1.0 + jnp.exp(-h2))
    o_ref[:] = h2


_CONV_BLOCK = 1024


def _mm_stats_body(n, bn, m_ref, w_ref, b_ref, h2_ref, st_ref):
    i = pl.program_id(0)
    h2 = _dot(m_ref[:], w_ref[:], ((1,), (1,))) + b_ref[:]
    h2_ref[:] = h2
    rows = lax.broadcasted_iota(jnp.int32, (bn, 1), 0)
    valid = rows < (n - i * bn)
    h2m = jnp.where(valid, h2, 0.0)
    s = jnp.sum(h2m, axis=0, keepdims=True)
    s2 = jnp.sum(h2m * h2m, axis=0, keepdims=True)

    @pl.when(i == 0)
    def _():
        st_ref[:] = jnp.zeros_like(st_ref)

    st_ref[0:1, :] += s
    st_ref[1:2, :] += s2


def _bn_act_body(n, sigmoid, h2_ref, st_ref, g_ref, be_ref, o_ref):
    mu = st_ref[0:1, :] / n
    var = st_ref[1:2, :] / n - mu * mu
    h2 = (h2_ref[:] - mu) * lax.rsqrt(var + 1e-5) * g_ref[:] + be_ref[:]
    h2 = jnp.where(h2 >= 0, h2, 0.2 * h2)
    if sigmoid:
        h2 = 1.0 / (1.0 + jnp.exp(-h2))
    o_ref[:] = h2


def _conv_bn(mat, W, b, g, be, sigmoid=False):
    n, kdim = mat.shape
    ocp = W.shape[0]
    b2, g2, be2 = b.reshape(1, -1), g.reshape(1, -1), be.reshape(1, -1)
    if n <= 2562:
        return pl.pallas_call(
            functools.partial(_conv_bn_body, sigmoid),
            out_shape=jax.ShapeDtypeStruct((n, ocp), jnp.float32),
        )(mat, W, b2, g2, be2)
    bn = _CONV_BLOCK
    nb = -(-n // bn)
    h2, st = pl.pallas_call(
        functools.partial(_mm_stats_body, n, bn),
        grid=(nb,),
        in_specs=[
            pl.BlockSpec((bn, kdim), lambda i: (i, 0)),
            pl.BlockSpec((ocp, kdim), lambda i: (0, 0)),
            pl.BlockSpec((1, ocp), lambda i: (0, 0)),
        ],
        out_specs=[
            pl.BlockSpec((bn, ocp), lambda i: (i, 0)),
            pl.BlockSpec((8, ocp), lambda i: (0, 0)),
        ],
        out_shape=[
            jax.ShapeDtypeStruct((n, ocp), jnp.float32),
            jax.ShapeDtypeStruct((8, ocp), jnp.float32),
        ],
    )(mat, W, b2)
    return pl.pallas_call(
        functools.partial(_bn_act_body, float(n), sigmoid),
        grid=(nb,),
        in_specs=[
            pl.BlockSpec((bn, ocp), lambda i: (i, 0)),
            pl.BlockSpec((8, ocp), lambda i: (0, 0)),
            pl.BlockSpec((1, ocp), lambda i: (0, 0)),
            pl.BlockSpec((1, ocp), lambda i: (0, 0)),
        ],
        out_specs=pl.BlockSpec((bn, ocp), lambda i: (i, 0)),
        out_shape=jax.ShapeDtypeStruct((n, ocp), jnp.float32),
    )(h2, st, g2, be2)


def _upconv_body(h_ref, w_ref, b_ref, o_ref):
    o_ref[:] = _dot(h_ref[:], w_ref[:], ((1,), (1,))) + b_ref[:]


def _upconv(h, W, b):
    return pl.pallas_call(
        _upconv_body,
        out_shape=jax.ShapeDtypeStruct((h.shape[0], W.shape[0]), jnp.float32),
    )(h, W, b.reshape(1, -1))


def _pad_cols(a, cp):
    return a if a.shape[1] == cp else jnp.pad(a, ((0, 0), (0, cp - a.shape[1])))


def _pad_conv_params(params, name, ic, icp, oc, ocp):
    W = params[name + '_W']
    W = W.reshape(oc, 7, ic)
    W = jnp.pad(W, ((0, ocp - oc), (0, 0), (0, icp - ic))).reshape(ocp, 7 * icp)
    b = jnp.pad(params[name + '_b'], (0, ocp - oc))
    g = jnp.pad(params[name + '_g'], (0, ocp - oc))
    be = jnp.pad(params[name + '_be'], (0, ocp - oc))
    return W, b, g, be


def _conv_layer(h, no, params, name, ic, icp, oc, ocp, sigmoid=False):
    n = no.shape[0] // 7
    W, b, g, be = _pad_conv_params(params, name, ic, icp, oc, ocp)
    gth = _gather(h, no)
    mat = gth.reshape(n, 7 * icp)
    return _conv_bn(mat, W, b, g, be, sigmoid=sigmoid)


def kernel(x, params, idx):
    h = _pad_cols(x, _CHP[0])
    for i in range(1, 6):
        n = _NS[i]
        hp = _gather_mean(h, idx['no%d' % (i - 1)][: n * 7], 7)
        h = _conv_layer(hp, idx['no%d' % i], params, 'd%dc1' % i,
                        _CHS[i - 1], _CHP[i - 1], _CHS[i], _CHP[i])
        h = _conv_layer(h, idx['no%d' % i], params, 'd%dc2' % i,
                        _CHS[i], _CHP[i], _CHS[i], _CHP[i])
    for (i, nf, nt, lvl) in _UPS:
        ic = _CHS[lvl + 1]
        oc = _UOC[i]
        ocp = 16 if oc == 3 else oc
        Wu = params['u%d_W' % i].reshape(7, oc, ic)
        Wu = jnp.pad(Wu, ((0, 0), (0, ocp - oc), (0, 0))).reshape(7 * ocp, ic)
        bu = jnp.pad(params['u%d_b' % i].reshape(7, oc),
                     ((0, 0), (0, ocp - oc))).reshape(7 * ocp)
        y = _upconv(h, Wu, bu)
        ytab = y.reshape(nf * 7, ocp)
        comb = jnp.concatenate(
            [jnp.repeat(idx['top%d' % nt], 2), idx['down%d' % nt]])
        h = _gather_mean(ytab, comb, 2)
        h = _conv_layer(h, idx['no%d' % lvl], params, 'u%dc1' % i,
                        oc, ocp, oc, ocp)
        h = _conv_layer(h, idx['no%d' % lvl], params, 'u%dc2' % i,
                        oc, ocp, oc, ocp, sigmoid=(i == 5))
    return h[:, :3]
